# baseline (speedup 1.0000x reference)
import functools

import jax
import jax.numpy as jnp
from jax import lax
from jax.experimental import pallas as pl
from jax.experimental.pallas import tpu as pltpu
from jax.experimental.pallas import tpu_sc as plsc

MODEL_DIM = 128
MLP_DIM = 256
NUM_CLASSES = 3
B = 16
L = 2048
NUM_WORKERS = 32
GCHUNK = 128
BLK = 8
SEG0_STEPS = 512
SEG1_STEPS = L - SEG0_STEPS
CHUNK_A = 256
CHUNK_B = 512


def _build_idx(tokens_hbm, tok_v, idx_v, goff):
    t0 = goff // B
    p0 = L - 8 - t0
    pltpu.sync_copy(tokens_hbm.at[pl.ds(p0 * B, 8 * B)], tok_v)
    for tl in range(8):
        idx_v[pl.ds(tl * B, B)] = tok_v[pl.ds((7 - tl) * B, B)]


def _gather_body(tokens_hbm, table_hbm, out_hbm, tok_v, idx_a, idx_b,
                 rows_a, rows_b, gsem_a, gsem_b, wsem_a, wsem_b, *,
                 rows_per_worker, row_base):
    wid = lax.axis_index("s") * 2 + lax.axis_index("c")
    base = wid * rows_per_worker
    nchunk = rows_per_worker // GCHUNK
    idx = [idx_a, idx_b]
    rows = [rows_a, rows_b]
    gsem = [gsem_a, gsem_b]
    wsem = [wsem_a, wsem_b]
    gcopy = [None, None]
    wcopy = [None, None]

    _build_idx(tokens_hbm, tok_v, idx_a, row_base + base)
    gcopy[0] = pltpu.async_copy(table_hbm.at[idx_a], rows_a, gsem_a)
    for c in range(nchunk):
        cur = c % 2
        nxt = 1 - cur
        if c + 1 < nchunk:
            off_n = base + (c + 1) * GCHUNK
            _build_idx(tokens_hbm, tok_v, idx[nxt], row_base + off_n)
            if wcopy[nxt] is not None:
                wcopy[nxt].wait()
            gcopy[nxt] = pltpu.async_copy(table_hbm.at[idx[nxt]], rows[nxt],
                                          gsem[nxt])
        gcopy[cur].wait()
        wcopy[cur] = pltpu.async_copy(
            rows[cur], out_hbm.at[pl.ds(base + c * GCHUNK, GCHUNK)],
            wsem[cur])
    for w in wcopy:
        if w is not None:
            w.wait()


@functools.cache
def _make_sc_gather(n_rows, row_base):
    body = functools.partial(_gather_body,
                             rows_per_worker=n_rows // NUM_WORKERS,
                             row_base=row_base)
    return pl.kernel(
        body,
        out_type=jax.ShapeDtypeStruct((n_rows, MODEL_DIM), jnp.float32),
        mesh=plsc.VectorSubcoreMesh(core_axis_name="c", subcore_axis_name="s",
                                    num_cores=2, num_subcores=16),
        scratch_types=[
            pltpu.VMEM((8 * B,), jnp.int32),
            pltpu.VMEM((GCHUNK,), jnp.int32),
            pltpu.VMEM((GCHUNK,), jnp.int32),
            pltpu.VMEM((GCHUNK, MODEL_DIM), jnp.float32),
            pltpu.VMEM((GCHUNK, MODEL_DIM), jnp.float32),
            pltpu.SemaphoreType.DMA,
            pltpu.SemaphoreType.DMA,
            pltpu.SemaphoreType.DMA,
            pltpu.SemaphoreType.DMA,
        ],
    )


_DG_T = (((1,), (1,)), ((), ()))


def _dot_t(a, b):
    return lax.dot_general(a, b, _DG_T, preferred_element_type=jnp.float32)


def _scan_chunk(x_ref, wih_ref, whh_ref, bih_ref, bhh_ref, act_ref, m_ref,
                h_ref, pre_ref, t_base, chunk):
    @pl.when(m_ref[0] < t_base + chunk)
    def _active():
        pre_ref[...] = (_dot_t(x_ref[...], wih_ref[...])
                        + (bih_ref[...] + bhh_ref[...]))
        whh = whh_ref[...]
        act = act_ref[...]
        nblk0 = jnp.clip((m_ref[0] - t_base) // BLK, 0, chunk // BLK)

        def blk(kb, h):
            for jj in range(BLK):
                j = kb * BLK + jj
                x = pre_ref[pl.ds(j * B, B), :]
                h_new = jnp.tanh(
                    x + jnp.dot(h, whh, preferred_element_type=jnp.float32))
                h = jnp.where(act <= t_base + j, h_new, h)
            return h

        h_ref[...] = lax.fori_loop(nblk0, chunk // BLK, blk, h_ref[...])


def _scan_a_body(x_ref, wih_ref, whh_ref, bih_ref, bhh_ref, act_ref, m_ref,
                 out_ref, h_ref, pre_ref):
    g = pl.program_id(0)
    ng = pl.num_programs(0)

    @pl.when(g == 0)
    def _init():
        h_ref[...] = jnp.zeros_like(h_ref)

    _scan_chunk(x_ref, wih_ref, whh_ref, bih_ref, bhh_ref, act_ref, m_ref,
                h_ref, pre_ref, g * CHUNK_A, CHUNK_A)

    @pl.when(g == ng - 1)
    def _emit():
        out_ref[...] = h_ref[...]


def _scan_b_body(x_ref, wih_ref, whh_ref, bih_ref, bhh_ref, act_ref, m_ref,
                 h_in_ref, w0_ref, b0_ref, w1_ref, b1_ref, out_ref, h_ref,
                 pre_ref):
    g = pl.program_id(0)
    ng = pl.num_programs(0)

    @pl.when(g == 0)
    def _init():
        h_ref[...] = h_in_ref[...]

    _scan_chunk(x_ref, wih_ref, whh_ref, bih_ref, bhh_ref, act_ref, m_ref,
                h_ref, pre_ref, SEG0_STEPS + g * CHUNK_B, CHUNK_B)

    @pl.when(g == ng - 1)
    def _mlp():
        h1 = jnp.maximum(_dot_t(h_ref[...], w0_ref[...]) + b0_ref[...], 0.0)
        h2 = jnp.maximum(_dot_t(h1, w1_ref[...]) + b1_ref[...], 0.0)
        lane = lax.broadcasted_iota(jnp.int32, (B, MODEL_DIM), 1)
        logits = jnp.where(lane < NUM_CLASSES, h2, -1e30)
        m = jnp.max(logits, axis=-1, keepdims=True)
        s = logits - m
        y = s - jnp.log(jnp.sum(jnp.exp(s), axis=-1, keepdims=True))
        out_ref[...] = y[:, :NUM_CLASSES]


def _full(shape):
    return pl.BlockSpec(shape, lambda g: tuple(0 for _ in shape))


def _common_specs(chunk):
    return [
        pl.BlockSpec((chunk * B, MODEL_DIM), lambda g: (g, 0)),
        _full((MODEL_DIM, MODEL_DIM)),
        _full((MODEL_DIM, MODEL_DIM)),
        _full((1, MODEL_DIM)),
        _full((1, MODEL_DIM)),
        _full((B, MODEL_DIM)),
        pl.BlockSpec(memory_space=pltpu.SMEM),
    ]


def _scratch(chunk):
    return [pltpu.VMEM((B, MODEL_DIM), jnp.float32),
            pltpu.VMEM((chunk * B, MODEL_DIM), jnp.float32)]


_scan_a_call = pl.pallas_call(
    _scan_a_body,
    grid=(SEG0_STEPS // CHUNK_A,),
    in_specs=_common_specs(CHUNK_A),
    out_specs=_full((B, MODEL_DIM)),
    out_shape=jax.ShapeDtypeStruct((B, MODEL_DIM), jnp.float32),
    scratch_shapes=_scratch(CHUNK_A),
)

_scan_b_call = pl.pallas_call(
    _scan_b_body,
    grid=(SEG1_STEPS // CHUNK_B,),
    in_specs=_common_specs(CHUNK_B) + [
        _full((B, MODEL_DIM)),
        _full((MLP_DIM, MODEL_DIM)),
        _full((1, MLP_DIM)),
        _full((MODEL_DIM, MLP_DIM)),
        _full((1, MODEL_DIM)),
    ],
    out_specs=_full((B, NUM_CLASSES)),
    out_shape=jax.ShapeDtypeStruct((B, NUM_CLASSES), jnp.float32),
    scratch_shapes=_scratch(CHUNK_B),
)


@jax.jit
def kernel(tokens, lengths, embeddings, W_ih, b_ih, W_hh, b_hh, W0, b0,
           W1, b1):
    tokens_t = tokens.T.reshape(-1)
    seg0 = SEG0_STEPS * B
    x0 = _make_sc_gather(seg0, 0)(tokens_t, embeddings)
    x1 = _make_sc_gather(SEG1_STEPS * B, seg0)(tokens_t, embeddings)

    act = jnp.broadcast_to((L - lengths)[:, None], (B, MODEL_DIM))
    act = act.astype(jnp.int32)
    b0r = b0[None, :]
    w1p = jnp.zeros((MODEL_DIM, MLP_DIM), W1.dtype).at[:NUM_CLASSES].set(W1)
    b1p = jnp.zeros((1, MODEL_DIM), b1.dtype).at[0, :NUM_CLASSES].set(b1)
    m = (L - jnp.max(lengths)).astype(jnp.int32).reshape(1)

    whh_t = W_hh.T
    h_mid = _scan_a_call(x0, W_ih, whh_t, b_ih[None, :], b_hh[None, :], act,
                         m)
    return _scan_b_call(x1, W_ih, whh_t, b_ih[None, :], b_hh[None, :], act,
                        m, h_mid, W0, b0r, w1p, b1p)


# speedup vs baseline: 1.0016x; 1.0016x over previous
import functools

import jax
import jax.numpy as jnp
from jax import lax
from jax.experimental import pallas as pl
from jax.experimental.pallas import tpu as pltpu
from jax.experimental.pallas import tpu_sc as plsc

MODEL_DIM = 128
MLP_DIM = 256
NUM_CLASSES = 3
B = 16
L = 2048
NUM_WORKERS = 32
GCHUNK = 128
BLK = 8
SEG0_STEPS = 256
SEG1_STEPS = L - SEG0_STEPS
CHUNK_A = 256
CHUNK_B = 448


def _build_idx(tokens_hbm, tok_v, idx_v, goff):
    t0 = goff // B
    p0 = L - 8 - t0
    pltpu.sync_copy(tokens_hbm.at[pl.ds(p0 * B, 8 * B)], tok_v)
    for tl in range(8):
        idx_v[pl.ds(tl * B, B)] = tok_v[pl.ds((7 - tl) * B, B)]


def _gather_body(tokens_hbm, table_hbm, out_hbm, tok_v, idx_a, idx_b,
                 rows_a, rows_b, gsem_a, gsem_b, wsem_a, wsem_b, *,
                 rows_per_worker, row_base):
    wid = lax.axis_index("s") * 2 + lax.axis_index("c")
    base = wid * rows_per_worker
    nchunk = rows_per_worker // GCHUNK
    idx = [idx_a, idx_b]
    rows = [rows_a, rows_b]
    gsem = [gsem_a, gsem_b]
    wsem = [wsem_a, wsem_b]
    gcopy = [None, None]
    wcopy = [None, None]

    _build_idx(tokens_hbm, tok_v, idx_a, row_base + base)
    gcopy[0] = pltpu.async_copy(table_hbm.at[idx_a], rows_a, gsem_a)
    for c in range(nchunk):
        cur = c % 2
        nxt = 1 - cur
        if c + 1 < nchunk:
            off_n = base + (c + 1) * GCHUNK
            _build_idx(tokens_hbm, tok_v, idx[nxt], row_base + off_n)
            if wcopy[nxt] is not None:
                wcopy[nxt].wait()
            gcopy[nxt] = pltpu.async_copy(table_hbm.at[idx[nxt]], rows[nxt],
                                          gsem[nxt])
        gcopy[cur].wait()
        wcopy[cur] = pltpu.async_copy(
            rows[cur], out_hbm.at[pl.ds(base + c * GCHUNK, GCHUNK)],
            wsem[cur])
    for w in wcopy:
        if w is not None:
            w.wait()


@functools.cache
def _make_sc_gather(n_rows, row_base):
    body = functools.partial(_gather_body,
                             rows_per_worker=n_rows // NUM_WORKERS,
                             row_base=row_base)
    return pl.kernel(
        body,
        out_type=jax.ShapeDtypeStruct((n_rows, MODEL_DIM), jnp.float32),
        mesh=plsc.VectorSubcoreMesh(core_axis_name="c", subcore_axis_name="s",
                                    num_cores=2, num_subcores=16),
        scratch_types=[
            pltpu.VMEM((8 * B,), jnp.int32),
            pltpu.VMEM((GCHUNK,), jnp.int32),
            pltpu.VMEM((GCHUNK,), jnp.int32),
            pltpu.VMEM((GCHUNK, MODEL_DIM), jnp.float32),
            pltpu.VMEM((GCHUNK, MODEL_DIM), jnp.float32),
            pltpu.SemaphoreType.DMA,
            pltpu.SemaphoreType.DMA,
            pltpu.SemaphoreType.DMA,
            pltpu.SemaphoreType.DMA,
        ],
    )


_DG_T = (((1,), (1,)), ((), ()))


def _dot_t(a, b):
    return lax.dot_general(a, b, _DG_T, preferred_element_type=jnp.float32)


def _scan_chunk(x_ref, wih_ref, whh_ref, bih_ref, bhh_ref, act_ref, m_ref,
                h_ref, pre_ref, t_base, chunk):
    @pl.when(m_ref[0] < t_base + chunk)
    def _active():
        pre_ref[...] = (_dot_t(x_ref[...], wih_ref[...])
                        + (bih_ref[...] + bhh_ref[...]))
        whh = whh_ref[...]
        act = act_ref[...]
        nblk0 = jnp.clip((m_ref[0] - t_base) // BLK, 0, chunk // BLK)

        def blk(kb, h):
            for jj in range(BLK):
                j = kb * BLK + jj
                x = pre_ref[pl.ds(j * B, B), :]
                h_new = jnp.tanh(
                    x + jnp.dot(h, whh, preferred_element_type=jnp.float32))
                h = jnp.where(act <= t_base + j, h_new, h)
            return h

        h_ref[...] = lax.fori_loop(nblk0, chunk // BLK, blk, h_ref[...])


def _scan_a_body(x_ref, wih_ref, whh_ref, bih_ref, bhh_ref, act_ref, m_ref,
                 out_ref, h_ref, pre_ref):
    g = pl.program_id(0)
    ng = pl.num_programs(0)

    @pl.when(g == 0)
    def _init():
        h_ref[...] = jnp.zeros_like(h_ref)

    _scan_chunk(x_ref, wih_ref, whh_ref, bih_ref, bhh_ref, act_ref, m_ref,
                h_ref, pre_ref, g * CHUNK_A, CHUNK_A)

    @pl.when(g == ng - 1)
    def _emit():
        out_ref[...] = h_ref[...]


def _scan_b_body(x_ref, wih_ref, whh_ref, bih_ref, bhh_ref, act_ref, m_ref,
                 h_in_ref, w0_ref, b0_ref, w1_ref, b1_ref, out_ref, h_ref,
                 pre_ref):
    g = pl.program_id(0)
    ng = pl.num_programs(0)

    @pl.when(g == 0)
    def _init():
        h_ref[...] = h_in_ref[...]

    _scan_chunk(x_ref, wih_ref, whh_ref, bih_ref, bhh_ref, act_ref, m_ref,
                h_ref, pre_ref, SEG0_STEPS + g * CHUNK_B, CHUNK_B)

    @pl.when(g == ng - 1)
    def _mlp():
        h1 = jnp.maximum(_dot_t(h_ref[...], w0_ref[...]) + b0_ref[...], 0.0)
        h2 = jnp.maximum(_dot_t(h1, w1_ref[...]) + b1_ref[...], 0.0)
        lane = lax.broadcasted_iota(jnp.int32, (B, MODEL_DIM), 1)
        logits = jnp.where(lane < NUM_CLASSES, h2, -1e30)
        m = jnp.max(logits, axis=-1, keepdims=True)
        s = logits - m
        y = s - jnp.log(jnp.sum(jnp.exp(s), axis=-1, keepdims=True))
        out_ref[...] = y[:, :NUM_CLASSES]


def _full(shape):
    return pl.BlockSpec(shape, lambda g: tuple(0 for _ in shape))


def _common_specs(chunk):
    return [
        pl.BlockSpec((chunk * B, MODEL_DIM), lambda g: (g, 0)),
        _full((MODEL_DIM, MODEL_DIM)),
        _full((MODEL_DIM, MODEL_DIM)),
        _full((1, MODEL_DIM)),
        _full((1, MODEL_DIM)),
        _full((B, MODEL_DIM)),
        pl.BlockSpec(memory_space=pltpu.SMEM),
    ]


def _scratch(chunk):
    return [pltpu.VMEM((B, MODEL_DIM), jnp.float32),
            pltpu.VMEM((chunk * B, MODEL_DIM), jnp.float32)]


_scan_a_call = pl.pallas_call(
    _scan_a_body,
    grid=(SEG0_STEPS // CHUNK_A,),
    in_specs=_common_specs(CHUNK_A),
    out_specs=_full((B, MODEL_DIM)),
    out_shape=jax.ShapeDtypeStruct((B, MODEL_DIM), jnp.float32),
    scratch_shapes=_scratch(CHUNK_A),
)

_scan_b_call = pl.pallas_call(
    _scan_b_body,
    grid=(SEG1_STEPS // CHUNK_B,),
    in_specs=_common_specs(CHUNK_B) + [
        _full((B, MODEL_DIM)),
        _full((MLP_DIM, MODEL_DIM)),
        _full((1, MLP_DIM)),
        _full((MODEL_DIM, MLP_DIM)),
        _full((1, MODEL_DIM)),
    ],
    out_specs=_full((B, NUM_CLASSES)),
    out_shape=jax.ShapeDtypeStruct((B, NUM_CLASSES), jnp.float32),
    scratch_shapes=_scratch(CHUNK_B),
)


@jax.jit
def kernel(tokens, lengths, embeddings, W_ih, b_ih, W_hh, b_hh, W0, b0,
           W1, b1):
    tokens_t = tokens.T.reshape(-1)
    seg0 = SEG0_STEPS * B
    x0 = _make_sc_gather(seg0, 0)(tokens_t, embeddings)
    x1 = _make_sc_gather(SEG1_STEPS * B, seg0)(tokens_t, embeddings)

    act = jnp.broadcast_to((L - lengths)[:, None], (B, MODEL_DIM))
    act = act.astype(jnp.int32)
    b0r = b0[None, :]
    w1p = jnp.zeros((MODEL_DIM, MLP_DIM), W1.dtype).at[:NUM_CLASSES].set(W1)
    b1p = jnp.zeros((1, MODEL_DIM), b1.dtype).at[0, :NUM_CLASSES].set(b1)
    m = (L - jnp.max(lengths)).astype(jnp.int32).reshape(1)

    whh_t = W_hh.T
    h_mid = _scan_a_call(x0, W_ih, whh_t, b_ih[None, :], b_hh[None, :], act,
                         m)
    return _scan_b_call(x1, W_ih, whh_t, b_ih[None, :], b_hh[None, :], act,
                        m, h_mid, W0, b0r, w1p, b1p)
